# baseline (speedup 1.0000x reference)
import functools

import jax
import jax.numpy as jnp
from jax import lax
from jax.experimental import pallas as pl
from jax.experimental.pallas import tpu as pltpu, tpu_sc as plsc

B = 8
N = 50000
E = 1600000
N_PAD = 50048
QP = N_PAD // 4
EPT = E // 4
C = 4000
NCHUNK = EPT // C
UNROLL = 10
NS = 16

_mesh = plsc.VectorSubcoreMesh(core_axis_name="c", subcore_axis_name="s")


@functools.partial(
    pl.kernel,
    out_type=(
        jax.ShapeDtypeStruct((B * N_PAD,), jnp.float32),
        jax.ShapeDtypeStruct((2 * NS * N_PAD,), jnp.float32),
    ),
    mesh=_mesh,
    compiler_params=pltpu.CompilerParams(
        needs_layout_passes=False, use_tc_tiling_on_sc=False
    ),
    scratch_types=[
        pltpu.VMEM((N_PAD,), jnp.float32),
        pltpu.VMEM((N_PAD,), jnp.float32),
        pltpu.VMEM((C,), jnp.int32),
        pltpu.VMEM((C,), jnp.int32),
        pltpu.VMEM((C,), jnp.float32),
        pltpu.VMEM((QP,), jnp.float32),
        pltpu.VMEM((16,), jnp.float32),
    ],
)
def _advect(rho_hbm, u_hbm, src_hbm, dst_hbm, params_hbm, out_hbm, part_hbm,
            rho_v, acc_v, src_v, dst_v, u_v, tmp_v, params_v):
    c = lax.axis_index("c")
    s = lax.axis_index("s")
    b_local = s % 4
    part = s // 4
    b = 4 * c + b_local

    pltpu.sync_copy(rho_hbm.at[b], rho_v)
    pltpu.sync_copy(params_hbm, params_v)
    kappa_vec = plsc.load_gather(params_v, [jnp.full((16,), 8, jnp.int32)])
    dt_vec = plsc.load_gather(params_v, [jnp.full((16,), b, jnp.int32)])

    zeros16 = jnp.zeros((16,), jnp.float32)

    def zero_body(j, carry):
        base = pl.multiple_of(j * 128, 128)
        for q in range(8):
            acc_v[pl.ds(base + q * 16, 16)] = zeros16
        return carry

    lax.fori_loop(0, N_PAD // 128, zero_body, 0)

    ebase = part * EPT

    def chunk_body(t, carry):
        off = t * C
        pltpu.sync_copy(src_hbm.at[pl.ds(ebase + off, C)], src_v)
        pltpu.sync_copy(dst_hbm.at[pl.ds(ebase + off, C)], dst_v)
        pltpu.sync_copy(u_hbm.at[b, pl.ds(ebase + off, C)], u_v)

        def step(i, inner):
            o = pl.multiple_of(i * (16 * UNROLL), 16)
            for q in range(UNROLL):
                oq = o + q * 16
                si = src_v[pl.ds(oq, 16)]
                di = dst_v[pl.ds(oq, 16)]
                uu = u_v[pl.ds(oq, 16)]
                rs = plsc.load_gather(rho_v, [si])
                rd = plsc.load_gather(rho_v, [di])
                up = jnp.where(uu >= 0.0, rs, rd)
                w = uu * up + kappa_vec * (rs - rd)
                plsc.addupdate_scatter(acc_v, [si], w)
                plsc.addupdate_scatter(acc_v, [di], -w)
            return inner

        lax.fori_loop(0, C // (16 * UNROLL), step, 0)
        return carry

    lax.fori_loop(0, NCHUNK, chunk_body, 0)

    tid = c * NS + s
    pltpu.sync_copy(acc_v, part_hbm.at[pl.ds(pl.multiple_of(tid * N_PAD, 8), N_PAD)])
    plsc.subcore_barrier()

    nbase = part * QP
    for p in range(4):
        j = c * NS + b_local + 4 * p

        @pl.when(j != tid)
        def _():
            pltpu.sync_copy(
                part_hbm.at[pl.ds(pl.multiple_of(j * N_PAD + nbase, 8), QP)], tmp_v
            )

            def add_body(i, carry):
                o = pl.multiple_of(i * 16, 16)
                acc_v[pl.ds(nbase + o, 16)] = (
                    acc_v[pl.ds(nbase + o, 16)] + tmp_v[pl.ds(o, 16)]
                )
                return carry

            lax.fori_loop(0, QP // 16, add_body, 0)

    def out_body(i, carry):
        o = pl.multiple_of(i * 16, 16)
        tmp_v[pl.ds(o, 16)] = (
            rho_v[pl.ds(nbase + o, 16)] - dt_vec * acc_v[pl.ds(nbase + o, 16)]
        )
        return carry

    lax.fori_loop(0, QP // 16, out_body, 0)
    pltpu.sync_copy(tmp_v, out_hbm.at[pl.ds(b * N_PAD + nbase, QP)])


def kernel(rho, u, dt, edge_src, edge_dst, kappa):
    rho_p = jnp.pad(rho, ((0, 0), (0, N_PAD - N)))
    params = (
        jnp.zeros((16,), jnp.float32)
        .at[:8].set(dt.astype(jnp.float32))
        .at[8].set(kappa.astype(jnp.float32))
    )
    out, _ = _advect(
        rho_p,
        u,
        edge_src.astype(jnp.int32),
        edge_dst.astype(jnp.int32),
        params,
    )
    return out.reshape(B, N_PAD)[:, :N]


# speedup vs baseline: 2.2626x; 2.2626x over previous
import functools

import jax
import jax.numpy as jnp
from jax import lax
from jax.experimental import pallas as pl
from jax.experimental.pallas import tpu as pltpu, tpu_sc as plsc

B = 8
N = 50000
E = 1600000
N_PAD = 50048
QP = N_PAD // 4
EPT = E // 4
C = 3200
CR = C // 128
NCHUNK = EPT // C
NS = 16

_mesh = plsc.VectorSubcoreMesh(core_axis_name="c", subcore_axis_name="s")


@functools.partial(
    pl.kernel,
    out_type=(
        jax.ShapeDtypeStruct((B * N_PAD,), jnp.float32),
        jax.ShapeDtypeStruct((2 * NS * N_PAD,), jnp.float32),
    ),
    mesh=_mesh,
    compiler_params=pltpu.CompilerParams(
        needs_layout_passes=False, use_tc_tiling_on_sc=False
    ),
    scratch_types=[
        pltpu.VMEM((N_PAD,), jnp.float32),
        pltpu.VMEM((N_PAD,), jnp.float32),
        pltpu.VMEM((C,), jnp.int32),
        pltpu.VMEM((C,), jnp.int32),
        pltpu.VMEM((CR, 128), jnp.float32),
        pltpu.VMEM((QP,), jnp.float32),
        pltpu.VMEM((16,), jnp.float32),
    ],
)
def _advect(rho_hbm, u_hbm, src_hbm, dst_hbm, params_hbm, out_hbm, part_hbm,
            rho_v, acc_v, src_v, dst_v, u_v, tmp_v, params_v):
    c = lax.axis_index("c")
    s = lax.axis_index("s")
    b_local = s % 4
    part = s // 4
    b = 4 * c + b_local

    pltpu.sync_copy(rho_hbm.at[b], rho_v)
    pltpu.sync_copy(params_hbm, params_v)
    kappa_vec = plsc.load_gather(params_v, [jnp.full((16,), 8, jnp.int32)])
    dt_vec = plsc.load_gather(params_v, [jnp.full((16,), b, jnp.int32)])

    zeros16 = jnp.zeros((16,), jnp.float32)

    def zero_body(j, carry):
        base = pl.multiple_of(j * 128, 128)
        for q in range(8):
            acc_v[pl.ds(base + q * 16, 16)] = zeros16
        return carry

    lax.fori_loop(0, N_PAD // 128, zero_body, 0)

    ebase = part * EPT

    def chunk_body(t, carry):
        off = t * C
        row0 = part * (EPT // 128) + t * CR
        pltpu.sync_copy(src_hbm.at[pl.ds(ebase + off, C)], src_v)
        pltpu.sync_copy(dst_hbm.at[pl.ds(ebase + off, C)], dst_v)
        pltpu.sync_copy(u_hbm.at[pl.ds(row0, CR), b, :], u_v)

        def step(r, inner):
            ro = pl.multiple_of(r * 128, 128)
            for q in range(8):
                oq = ro + q * 16
                si = src_v[pl.ds(oq, 16)]
                di = dst_v[pl.ds(oq, 16)]
                uu = u_v[r, pl.ds(q * 16, 16)]
                rs = plsc.load_gather(rho_v, [si])
                rd = plsc.load_gather(rho_v, [di])
                up = jnp.where(uu >= 0.0, rs, rd)
                w = uu * up + kappa_vec * (rs - rd)
                plsc.addupdate_scatter(acc_v, [si], w)
                plsc.addupdate_scatter(acc_v, [di], -w)
            return inner

        lax.fori_loop(0, CR, step, 0)
        return carry

    lax.fori_loop(0, NCHUNK, chunk_body, 0)

    tid = c * NS + s
    pltpu.sync_copy(acc_v, part_hbm.at[pl.ds(pl.multiple_of(tid * N_PAD, 8), N_PAD)])
    plsc.subcore_barrier()

    nbase = part * QP
    for p in range(4):
        j = c * NS + b_local + 4 * p

        @pl.when(j != tid)
        def _():
            pltpu.sync_copy(
                part_hbm.at[pl.ds(pl.multiple_of(j * N_PAD + nbase, 8), QP)], tmp_v
            )

            def add_body(i, carry):
                o = pl.multiple_of(i * 16, 16)
                acc_v[pl.ds(nbase + o, 16)] = (
                    acc_v[pl.ds(nbase + o, 16)] + tmp_v[pl.ds(o, 16)]
                )
                return carry

            lax.fori_loop(0, QP // 16, add_body, 0)

    def out_body(i, carry):
        o = pl.multiple_of(i * 16, 16)
        tmp_v[pl.ds(o, 16)] = (
            rho_v[pl.ds(nbase + o, 16)] - dt_vec * acc_v[pl.ds(nbase + o, 16)]
        )
        return carry

    lax.fori_loop(0, QP // 16, out_body, 0)
    pltpu.sync_copy(tmp_v, out_hbm.at[pl.ds(b * N_PAD + nbase, QP)])


def kernel(rho, u, dt, edge_src, edge_dst, kappa):
    rho_p = jnp.pad(rho, ((0, 0), (0, N_PAD - N)))
    params = (
        jnp.zeros((16,), jnp.float32)
        .at[:8].set(dt.astype(jnp.float32))
        .at[8].set(kappa.astype(jnp.float32))
    )
    u3 = u.reshape(B, E // 128, 128).transpose(1, 0, 2)
    out, _ = _advect(
        rho_p,
        u3,
        edge_src.astype(jnp.int32),
        edge_dst.astype(jnp.int32),
        params,
    )
    return out.reshape(B, N_PAD)[:, :N]


# speedup vs baseline: 7.7047x; 3.4052x over previous
import functools

import jax
import jax.numpy as jnp
from jax import lax
from jax.experimental import pallas as pl
from jax.experimental.pallas import tpu as pltpu, tpu_sc as plsc

B = 8
N = 50000
E = 1600000
N_PAD = 50048
QP = N_PAD // 4
HQ = QP // 2
EPT = E // 4
C = 3200
CR = C // 128
NCHUNK = EPT // C
NS = 16

_mesh = plsc.VectorSubcoreMesh(core_axis_name="c", subcore_axis_name="s")


@functools.partial(
    pl.kernel,
    out_type=(
        jax.ShapeDtypeStruct((B * N_PAD,), jnp.float32),
        jax.ShapeDtypeStruct((2 * NS * N_PAD,), jnp.float32),
    ),
    mesh=_mesh,
    compiler_params=pltpu.CompilerParams(
        needs_layout_passes=False, use_tc_tiling_on_sc=False
    ),
    scratch_types=[
        pltpu.VMEM((N_PAD,), jnp.float32),
        pltpu.VMEM((N_PAD,), jnp.float32),
        pltpu.VMEM((2, C), jnp.int32),
        pltpu.VMEM((2, C), jnp.int32),
        pltpu.VMEM((2, CR, 128), jnp.float32),
        pltpu.VMEM((HQ,), jnp.float32),
        pltpu.VMEM((16,), jnp.float32),
        pltpu.SemaphoreType.DMA,
        pltpu.SemaphoreType.DMA,
    ],
)
def _advect(rho_hbm, u_hbm, src_hbm, dst_hbm, params_hbm, out_hbm, part_hbm,
            rho_v, acc_v, src_d, dst_d, u_d, tmp_v, params_v, sem0, sem1):
    c = lax.axis_index("c")
    s = lax.axis_index("s")
    b_local = s % 4
    part = s // 4
    b = 4 * c + b_local
    sems = (sem0, sem1)

    pltpu.sync_copy(rho_hbm.at[pl.ds(b * N_PAD, N_PAD)], rho_v)
    pltpu.sync_copy(params_hbm, params_v)
    kappa_vec = plsc.load_gather(params_v, [jnp.full((16,), 8, jnp.int32)])
    dt_vec = plsc.load_gather(params_v, [jnp.full((16,), b, jnp.int32)])

    ebase = part * EPT
    rbase = part * (EPT // 128)

    def start_chunk(t, j):
        off = t * C
        pltpu.async_copy(src_hbm.at[pl.ds(ebase + off, C)], src_d.at[j], sems[j])
        pltpu.async_copy(dst_hbm.at[pl.ds(ebase + off, C)], dst_d.at[j], sems[j])
        pltpu.async_copy(u_hbm.at[pl.ds(rbase + t * CR, CR), b, :], u_d.at[j],
                         sems[j])

    def wait_chunk(j):
        pltpu.make_async_copy(src_hbm.at[pl.ds(0, C)], src_d.at[j], sems[j]).wait()
        pltpu.make_async_copy(dst_hbm.at[pl.ds(0, C)], dst_d.at[j], sems[j]).wait()
        pltpu.make_async_copy(u_hbm.at[pl.ds(0, CR), 0, :], u_d.at[j],
                              sems[j]).wait()

    start_chunk(0, 0)

    zeros16 = jnp.zeros((16,), jnp.float32)

    @plsc.parallel_loop(0, N_PAD // 128)
    def _zero(jj):
        base = pl.multiple_of(jj * 128, 128)
        for q in range(8):
            acc_v[pl.ds(base + q * 16, 16)] = zeros16

    def compute_chunk(j):
        @plsc.parallel_loop(0, CR)
        def _rows(r):
            ro = pl.multiple_of(r * 128, 128)
            for q in range(8):
                oq = ro + q * 16
                si = src_d[j, pl.ds(oq, 16)]
                di = dst_d[j, pl.ds(oq, 16)]
                uu = u_d[j, r, pl.ds(q * 16, 16)]
                rs = plsc.load_gather(rho_v, [si])
                rd = plsc.load_gather(rho_v, [di])
                up = jnp.where(uu >= 0.0, rs, rd)
                w = uu * up + kappa_vec * (rs - rd)
                plsc.addupdate_scatter(acc_v, [si], w)
                plsc.addupdate_scatter(acc_v, [di], -w)

    def outer(t0, carry):
        for j in range(2):
            t = t0 * 2 + j
            start_chunk(t + 1, 1 - j)
            wait_chunk(j)
            compute_chunk(j)
        return carry

    lax.fori_loop(0, (NCHUNK - 1) // 2, outer, 0)
    wait_chunk(0)
    compute_chunk(0)

    tid = c * NS + s
    pltpu.sync_copy(acc_v, part_hbm.at[pl.ds(pl.multiple_of(tid * N_PAD, 8), N_PAD)])
    plsc.subcore_barrier()

    nbase = part * QP
    for p in range(4):
        jj = c * NS + b_local + 4 * p

        @pl.when(jj != tid)
        def _():
            for blk in range(2):
                boff = nbase + blk * HQ
                pltpu.sync_copy(
                    part_hbm.at[pl.ds(pl.multiple_of(jj * N_PAD + boff, 8), HQ)],
                    tmp_v,
                )

                @plsc.parallel_loop(0, HQ // 16)
                def _add(i):
                    o = pl.multiple_of(i * 16, 16)
                    acc_v[pl.ds(boff + o, 16)] = (
                        acc_v[pl.ds(boff + o, 16)] + tmp_v[pl.ds(o, 16)]
                    )

    for blk in range(2):
        boff = nbase + blk * HQ

        @plsc.parallel_loop(0, HQ // 16)
        def _out(i):
            o = pl.multiple_of(i * 16, 16)
            tmp_v[pl.ds(o, 16)] = (
                rho_v[pl.ds(boff + o, 16)] - dt_vec * acc_v[pl.ds(boff + o, 16)]
            )

        pltpu.sync_copy(tmp_v, out_hbm.at[pl.ds(b * N_PAD + boff, HQ)])


def kernel(rho, u, dt, edge_src, edge_dst, kappa):
    rho_p = jnp.pad(rho, ((0, 0), (0, N_PAD - N))).reshape(-1)
    params = (
        jnp.zeros((16,), jnp.float32)
        .at[:8].set(dt.astype(jnp.float32))
        .at[8].set(kappa.astype(jnp.float32))
    )
    u3 = u.reshape(B, E // 128, 128).transpose(1, 0, 2)
    out, _ = _advect(
        rho_p,
        u3,
        edge_src.astype(jnp.int32),
        edge_dst.astype(jnp.int32),
        params,
    )
    return out.reshape(B, N_PAD)[:, :N]
